# baseline (speedup 1.0000x reference)
import jax
import jax.numpy as jnp
from jax import lax
from jax.experimental import pallas as pl
from jax.experimental.pallas import tpu as pltpu
from jax.experimental.pallas import tpu_sc as plsc

_N_CHAINS = 64
_N_SITES = 8192
_LANES = 16
_PER_WORKER = 2 * _N_SITES

_ROTS = (13, 15, 26, 6, 17, 29, 16, 24)


def _threefry2x32(k0, k1, x0, x1):
    ks = (k0, k1, k0 ^ k1 ^ jnp.int32(0x1BD11BDA))
    x0 = x0 + ks[0]
    x1 = x1 + ks[1]
    for g in range(5):
        for j in range(4):
            r = _ROTS[(g % 2) * 4 + j]
            x0 = x0 + x1
            x1 = (x1 << jnp.int32(r)) | lax.shift_right_logical(
                x1, jnp.int32(32 - r)
            )
            x1 = x0 ^ x1
        x0 = x0 + ks[(g + 1) % 3]
        x1 = x1 + ks[(g + 2) % 3] + jnp.int32(g + 1)
    return x0, x1


def _sc_body(x_hbm, seeds_hbm, out_hbm, seed_v, row_buf, sem_in, sem_seed,
             sem_out):
    q = lax.axis_index("c")
    s = lax.axis_index("s")
    w = q * 16 + s
    base = w * _PER_WORKER

    cin = pltpu.async_copy(x_hbm.at[pl.ds(base, _PER_WORKER)], row_buf, sem_in)

    grp = w >> 3
    pltpu.async_copy(
        seeds_hbm.at[pl.ds(grp * _LANES, _LANES)], seed_v, sem_seed
    ).wait()
    sv = seed_v[...]

    zero = jnp.zeros((_LANES,), jnp.int32)
    one = zero + jnp.int32(1)
    f0, f1 = _threefry2x32(zero, zero, zero, sv)
    k20, k21 = _threefry2x32(f0, f1, zero, one)
    y0, y1 = _threefry2x32(k20, k21, zero, zero)
    idx = (y0 ^ y1) & jnp.int32(_N_SITES - 1)

    lanes = lax.iota(jnp.int32, _LANES)
    l0 = (w & 7) * 2
    mask = (lanes >= l0) & (lanes < l0 + 2)
    flat = jnp.where(mask, (lanes - l0) * _N_SITES + idx, 0)

    cin.wait()
    vals = plsc.load_gather(row_buf, [flat], mask=mask)
    plsc.store_scatter(row_buf, [flat], -vals, mask=mask)
    pltpu.async_copy(row_buf, out_hbm.at[pl.ds(base, _PER_WORKER)], sem_out).wait()


_compiler_params = pltpu.CompilerParams(
    needs_layout_passes=False, use_tc_tiling_on_sc=False
)

_sc_call = pl.kernel(
    _sc_body,
    compiler_params=_compiler_params,
    out_type=jax.ShapeDtypeStruct((_N_CHAINS * _N_SITES,), jnp.float32),
    mesh=plsc.VectorSubcoreMesh(
        core_axis_name="c", subcore_axis_name="s", num_cores=2, num_subcores=16
    ),
    scratch_types=[
        pltpu.VMEM((_LANES,), jnp.int32),
        pltpu.VMEM((_PER_WORKER,), jnp.float32),
        pltpu.SemaphoreType.DMA,
        pltpu.SemaphoreType.DMA,
        pltpu.SemaphoreType.DMA,
    ],
)


def kernel(x, seeds):
    out = _sc_call(x.reshape(_N_CHAINS * _N_SITES), seeds)
    return out.reshape(_N_CHAINS, _N_SITES)


# speedup vs baseline: 3.8632x; 3.8632x over previous
import jax
import jax.numpy as jnp
from jax.experimental import pallas as pl
from jax.experimental.pallas import tpu as pltpu

_N_CHAINS = 64
_N_SITES = 8192
_BLK = 1024

_ROTS = (13, 15, 26, 6, 17, 29, 16, 24)


def _threefry2x32(k0, k1, x0, x1):
    ks = (k0, k1, k0 ^ k1 ^ jnp.uint32(0x1BD11BDA))
    x0 = x0 + ks[0]
    x1 = x1 + ks[1]
    for g in range(5):
        for j in range(4):
            r = _ROTS[(g % 2) * 4 + j]
            x0 = x0 + x1
            x1 = (x1 << jnp.uint32(r)) | (x1 >> jnp.uint32(32 - r))
            x1 = x0 ^ x1
        x0 = x0 + ks[(g + 1) % 3]
        x1 = x1 + ks[(g + 2) % 3] + jnp.uint32(g + 1)
    return x0, x1


def _flip_kernel(x_ref, seeds_ref, out_ref, idx_ref):
    j = pl.program_id(0)

    @pl.when(j == 0)
    def _rng():
        s = seeds_ref[...].astype(jnp.uint32)
        zero = jnp.zeros_like(s)
        one = jnp.ones_like(s)
        f0, f1 = _threefry2x32(zero, zero, zero, s)
        k20, k21 = _threefry2x32(f0, f1, zero, one)
        y0, y1 = _threefry2x32(k20, k21, zero, zero)
        bits = y0 ^ y1
        idx = (bits & jnp.uint32(_N_SITES - 1)).astype(jnp.int32)
        idx_ref[...] = idx.reshape(_N_CHAINS, 1)

    idx_col = idx_ref[...]
    col = jax.lax.broadcasted_iota(jnp.int32, (_N_CHAINS, _BLK), 1) + j * _BLK
    xv = x_ref[...]
    out_ref[...] = jnp.where(col == idx_col, -xv, xv)


def kernel(x, seeds):
    seeds2d = seeds.reshape(1, _N_CHAINS)
    return pl.pallas_call(
        _flip_kernel,
        grid=(_N_SITES // _BLK,),
        in_specs=[
            pl.BlockSpec((_N_CHAINS, _BLK), lambda j: (0, j)),
            pl.BlockSpec((1, _N_CHAINS), lambda j: (0, 0)),
        ],
        out_specs=pl.BlockSpec((_N_CHAINS, _BLK), lambda j: (0, j)),
        out_shape=jax.ShapeDtypeStruct((_N_CHAINS, _N_SITES), jnp.float32),
        scratch_shapes=[pltpu.VMEM((_N_CHAINS, 1), jnp.int32)],
    )(x, seeds2d)


# speedup vs baseline: 3.9183x; 1.0142x over previous
import jax
import jax.numpy as jnp
from jax.experimental import pallas as pl
from jax.experimental.pallas import tpu as pltpu

_N_CHAINS = 64
_N_SITES = 8192
_BLK_ROWS = 8

_ROTS = (13, 15, 26, 6, 17, 29, 16, 24)


def _threefry2x32(k0, k1, x0, x1):
    ks = (k0, k1, k0 ^ k1 ^ jnp.uint32(0x1BD11BDA))
    x0 = x0 + ks[0]
    x1 = x1 + ks[1]
    for g in range(5):
        for j in range(4):
            r = _ROTS[(g % 2) * 4 + j]
            x0 = x0 + x1
            x1 = (x1 << jnp.uint32(r)) | (x1 >> jnp.uint32(32 - r))
            x1 = x0 ^ x1
        x0 = x0 + ks[(g + 1) % 3]
        x1 = x1 + ks[(g + 2) % 3] + jnp.uint32(g + 1)
    return x0, x1


def _flip_kernel(x_ref, seeds_ref, out_ref, idx_ref):
    j = pl.program_id(0)

    @pl.when(j == 0)
    def _rng():
        s = seeds_ref[...].astype(jnp.uint32)
        zero = jnp.zeros_like(s)
        one = jnp.ones_like(s)
        f0, f1 = _threefry2x32(zero, zero, zero, s)
        k20, k21 = _threefry2x32(f0, f1, zero, one)
        y0, y1 = _threefry2x32(k20, k21, zero, zero)
        bits = y0 ^ y1
        idx = (bits & jnp.uint32(_N_SITES - 1)).astype(jnp.int32)
        idx_ref[...] = idx.reshape(_N_CHAINS, 1)

    idx_col = idx_ref[pl.ds(j * _BLK_ROWS, _BLK_ROWS), :]
    col = jax.lax.broadcasted_iota(jnp.int32, (_BLK_ROWS, _N_SITES), 1)
    xv = x_ref[...]
    out_ref[...] = jnp.where(col == idx_col, -xv, xv)


def kernel(x, seeds):
    seeds2d = seeds.reshape(1, _N_CHAINS)
    return pl.pallas_call(
        _flip_kernel,
        grid=(_N_CHAINS // _BLK_ROWS,),
        in_specs=[
            pl.BlockSpec((_BLK_ROWS, _N_SITES), lambda j: (j, 0)),
            pl.BlockSpec((1, _N_CHAINS), lambda j: (0, 0)),
        ],
        out_specs=pl.BlockSpec((_BLK_ROWS, _N_SITES), lambda j: (j, 0)),
        out_shape=jax.ShapeDtypeStruct((_N_CHAINS, _N_SITES), jnp.float32),
        scratch_shapes=[pltpu.VMEM((_N_CHAINS, 1), jnp.int32)],
    )(x, seeds2d)


# speedup vs baseline: 4.0591x; 1.0359x over previous
import jax
import jax.numpy as jnp
from jax.experimental import pallas as pl
from jax.experimental.pallas import tpu as pltpu

_N_CHAINS = 64
_N_SITES = 8192
_CHUNK = 8
_N_CHUNKS = _N_CHAINS // _CHUNK

_ROTS = (13, 15, 26, 6, 17, 29, 16, 24)


def _threefry2x32(k0, k1, x0, x1):
    ks = (k0, k1, k0 ^ k1 ^ jnp.uint32(0x1BD11BDA))
    x0 = x0 + ks[0]
    x1 = x1 + ks[1]
    for g in range(5):
        for j in range(4):
            r = _ROTS[(g % 2) * 4 + j]
            x0 = x0 + x1
            x1 = (x1 << jnp.uint32(r)) | (x1 >> jnp.uint32(32 - r))
            x1 = x0 ^ x1
        x0 = x0 + ks[(g + 1) % 3]
        x1 = x1 + ks[(g + 2) % 3] + jnp.uint32(g + 1)
    return x0, x1


def _flip_kernel(x_hbm, seeds_ref, out_hbm, in0, in1, ou0, ou1, insem, outsem):
    inbufs = (in0, in1)
    outbufs = (ou0, ou1)

    def in_dma(k):
        return pltpu.make_async_copy(
            x_hbm.at[pl.ds(k * _CHUNK, _CHUNK)], inbufs[k % 2], insem.at[k]
        )

    def out_dma(k):
        return pltpu.make_async_copy(
            outbufs[k % 2], out_hbm.at[pl.ds(k * _CHUNK, _CHUNK)], outsem.at[k]
        )

    in_dma(0).start()
    in_dma(1).start()

    s = seeds_ref[...].astype(jnp.uint32)
    zero = jnp.zeros_like(s)
    one = jnp.ones_like(s)
    f0, f1 = _threefry2x32(zero, zero, zero, s)
    k20, k21 = _threefry2x32(f0, f1, zero, one)
    y0, y1 = _threefry2x32(k20, k21, zero, zero)
    bits = y0 ^ y1
    idx = (bits & jnp.uint32(_N_SITES - 1)).astype(jnp.int32)
    idx_col = idx.reshape(_N_CHAINS, 1)

    col = jax.lax.broadcasted_iota(jnp.int32, (_CHUNK, _N_SITES), 1)

    for k in range(_N_CHUNKS):
        in_dma(k).wait()
        v = inbufs[k % 2][...]
        idx_k = jax.lax.slice(idx_col, (k * _CHUNK, 0), ((k + 1) * _CHUNK, 1))
        if k >= 2:
            out_dma(k - 2).wait()
        outbufs[k % 2][...] = jnp.where(col == idx_k, -v, v)
        out_dma(k).start()
        if k + 2 < _N_CHUNKS:
            in_dma(k + 2).start()

    out_dma(_N_CHUNKS - 2).wait()
    out_dma(_N_CHUNKS - 1).wait()


def kernel(x, seeds):
    seeds2d = seeds.reshape(1, _N_CHAINS)
    return pl.pallas_call(
        _flip_kernel,
        in_specs=[
            pl.BlockSpec(memory_space=pl.ANY),
            pl.BlockSpec((1, _N_CHAINS), lambda: (0, 0)),
        ],
        out_specs=pl.BlockSpec(memory_space=pl.ANY),
        out_shape=jax.ShapeDtypeStruct((_N_CHAINS, _N_SITES), jnp.float32),
        scratch_shapes=[
            pltpu.VMEM((_CHUNK, _N_SITES), jnp.float32),
            pltpu.VMEM((_CHUNK, _N_SITES), jnp.float32),
            pltpu.VMEM((_CHUNK, _N_SITES), jnp.float32),
            pltpu.VMEM((_CHUNK, _N_SITES), jnp.float32),
            pltpu.SemaphoreType.DMA((_N_CHUNKS,)),
            pltpu.SemaphoreType.DMA((_N_CHUNKS,)),
        ],
    )(x, seeds2d)


# speedup vs baseline: 7.3745x; 1.8168x over previous
import jax
import jax.numpy as jnp
from jax.experimental import pallas as pl
from jax.experimental.pallas import tpu as pltpu

_N_CHAINS = 64
_N_SITES = 8192
_CHUNK = 32
_N_CHUNKS = _N_CHAINS // _CHUNK

_ROTS = (13, 15, 26, 6, 17, 29, 16, 24)


def _threefry2x32(k0, k1, x0, x1):
    ks = (k0, k1, k0 ^ k1 ^ jnp.uint32(0x1BD11BDA))
    x0 = x0 + ks[0]
    x1 = x1 + ks[1]
    for g in range(5):
        for j in range(4):
            r = _ROTS[(g % 2) * 4 + j]
            x0 = x0 + x1
            x1 = (x1 << jnp.uint32(r)) | (x1 >> jnp.uint32(32 - r))
            x1 = x0 ^ x1
        x0 = x0 + ks[(g + 1) % 3]
        x1 = x1 + ks[(g + 2) % 3] + jnp.uint32(g + 1)
    return x0, x1


def _flip_kernel(x_hbm, seeds_ref, out_hbm, in0, in1, ou0, ou1, insem, outsem):
    inbufs = (in0, in1)
    outbufs = (ou0, ou1)

    def in_dma(k):
        return pltpu.make_async_copy(
            x_hbm.at[pl.ds(k * _CHUNK, _CHUNK)], inbufs[k % 2], insem.at[k]
        )

    def out_dma(k):
        return pltpu.make_async_copy(
            outbufs[k % 2], out_hbm.at[pl.ds(k * _CHUNK, _CHUNK)], outsem.at[k]
        )

    in_dma(0).start()
    in_dma(1).start()

    s = seeds_ref[...].astype(jnp.uint32)
    zero = jnp.zeros_like(s)
    one = jnp.ones_like(s)
    f0, f1 = _threefry2x32(zero, zero, zero, s)
    k20, k21 = _threefry2x32(f0, f1, zero, one)
    y0, y1 = _threefry2x32(k20, k21, zero, zero)
    bits = y0 ^ y1
    idx = (bits & jnp.uint32(_N_SITES - 1)).astype(jnp.int32)
    idx_col = idx.reshape(_N_CHAINS, 1)

    col = jax.lax.broadcasted_iota(jnp.int32, (_CHUNK, _N_SITES), 1)

    for k in range(_N_CHUNKS):
        in_dma(k).wait()
        v = inbufs[k % 2][...]
        idx_k = jax.lax.slice(idx_col, (k * _CHUNK, 0), ((k + 1) * _CHUNK, 1))
        if k >= 2:
            out_dma(k - 2).wait()
        outbufs[k % 2][...] = jnp.where(col == idx_k, -v, v)
        out_dma(k).start()
        if k + 2 < _N_CHUNKS:
            in_dma(k + 2).start()

    out_dma(_N_CHUNKS - 2).wait()
    out_dma(_N_CHUNKS - 1).wait()


def kernel(x, seeds):
    seeds2d = seeds.reshape(1, _N_CHAINS)
    return pl.pallas_call(
        _flip_kernel,
        in_specs=[
            pl.BlockSpec(memory_space=pl.ANY),
            pl.BlockSpec((1, _N_CHAINS), lambda: (0, 0)),
        ],
        out_specs=pl.BlockSpec(memory_space=pl.ANY),
        out_shape=jax.ShapeDtypeStruct((_N_CHAINS, _N_SITES), jnp.float32),
        scratch_shapes=[
            pltpu.VMEM((_CHUNK, _N_SITES), jnp.float32),
            pltpu.VMEM((_CHUNK, _N_SITES), jnp.float32),
            pltpu.VMEM((_CHUNK, _N_SITES), jnp.float32),
            pltpu.VMEM((_CHUNK, _N_SITES), jnp.float32),
            pltpu.SemaphoreType.DMA((_N_CHUNKS,)),
            pltpu.SemaphoreType.DMA((_N_CHUNKS,)),
        ],
    )(x, seeds2d)


# speedup vs baseline: 7.4438x; 1.0094x over previous
import jax
import jax.numpy as jnp
from jax.experimental import pallas as pl
from jax.experimental.pallas import tpu as pltpu

_N_CHAINS = 64
_N_SITES = 8192
_CHUNK = 32
_N_CHUNKS = _N_CHAINS // _CHUNK

_ROTS = (13, 15, 26, 6, 17, 29, 16, 24)


def _threefry2x32(k0, k1, x0, x1):
    ks = (k0, k1, k0 ^ k1 ^ jnp.uint32(0x1BD11BDA))
    x0 = x0 + ks[0]
    x1 = x1 + ks[1]
    for g in range(5):
        for j in range(4):
            r = _ROTS[(g % 2) * 4 + j]
            x0 = x0 + x1
            x1 = (x1 << jnp.uint32(r)) | (x1 >> jnp.uint32(32 - r))
            x1 = x0 ^ x1
        x0 = x0 + ks[(g + 1) % 3]
        x1 = x1 + ks[(g + 2) % 3] + jnp.uint32(g + 1)
    return x0, x1


def _flip_kernel(x_hbm, seeds_ref, out_hbm, in0, in1, ou0, ou1, insem, outsem):
    inbufs = (in0, in1)
    outbufs = (ou0, ou1)

    def in_dma(k):
        return pltpu.make_async_copy(
            x_hbm.at[pl.ds(k * _CHUNK, _CHUNK)], inbufs[k % 2], insem.at[k]
        )

    def out_dma(k):
        return pltpu.make_async_copy(
            outbufs[k % 2], out_hbm.at[pl.ds(k * _CHUNK, _CHUNK)], outsem.at[k]
        )

    in_dma(0).start()
    in_dma(1).start()

    s = seeds_ref[...].astype(jnp.uint32)
    zero = jnp.zeros_like(s)
    one = jnp.ones_like(s)
    f0, f1 = _threefry2x32(zero, zero, zero, s)
    k20, k21 = _threefry2x32(f0, f1, zero, one)
    y0, y1 = _threefry2x32(k20, k21, zero, zero)
    bits = y0 ^ y1
    idx = (bits & jnp.uint32(_N_SITES - 1)).astype(jnp.int32)
    idx_col = idx.reshape(_N_CHAINS, 1)

    col = jax.lax.broadcasted_iota(jnp.int32, (_CHUNK, _N_SITES), 1)

    sign_bit = jnp.full((_CHUNK, _N_SITES), jnp.int32(-2147483648), jnp.int32)
    zeros32 = jnp.zeros((_CHUNK, _N_SITES), jnp.int32)

    for k in range(_N_CHUNKS):
        in_dma(k).wait()
        v = inbufs[k % 2][...].view(jnp.int32)
        idx_k = jax.lax.slice(idx_col, (k * _CHUNK, 0), ((k + 1) * _CHUNK, 1))
        if k >= 2:
            out_dma(k - 2).wait()
        delta = jnp.where(col == idx_k, sign_bit, zeros32)
        outbufs[k % 2][...] = (v ^ delta).view(jnp.float32)
        out_dma(k).start()
        if k + 2 < _N_CHUNKS:
            in_dma(k + 2).start()

    out_dma(_N_CHUNKS - 2).wait()
    out_dma(_N_CHUNKS - 1).wait()


def kernel(x, seeds):
    seeds2d = seeds.reshape(1, _N_CHAINS)
    return pl.pallas_call(
        _flip_kernel,
        in_specs=[
            pl.BlockSpec(memory_space=pl.ANY),
            pl.BlockSpec((1, _N_CHAINS), lambda: (0, 0)),
        ],
        out_specs=pl.BlockSpec(memory_space=pl.ANY),
        out_shape=jax.ShapeDtypeStruct((_N_CHAINS, _N_SITES), jnp.float32),
        scratch_shapes=[
            pltpu.VMEM((_CHUNK, _N_SITES), jnp.float32),
            pltpu.VMEM((_CHUNK, _N_SITES), jnp.float32),
            pltpu.VMEM((_CHUNK, _N_SITES), jnp.float32),
            pltpu.VMEM((_CHUNK, _N_SITES), jnp.float32),
            pltpu.SemaphoreType.DMA((_N_CHUNKS,)),
            pltpu.SemaphoreType.DMA((_N_CHUNKS,)),
        ],
    )(x, seeds2d)
